# baseline (speedup 1.0000x reference)
import functools

import jax
import jax.numpy as jnp
from jax import lax
from jax.experimental import pallas as pl
from jax.experimental.pallas import tpu as pltpu
from jax.experimental.pallas import tpu_sc as plsc

_EMBED = 64
_SCALE = 8.0

_info = plsc.get_sparse_core_info()
_NC = _info.num_cores
_NS = _info.num_subcores
_L = _info.num_lanes
_NW = _NC * _NS

_VECS_PER_ROW = _EMBED // _L
_NBUF = 4
_AHEAD = 2
_ROW_UNROLL = 8


def kernel(tokens, table):
    S, T = tokens.shape
    tok = tokens.astype(jnp.int32)
    s_per_w = S // _NW
    t_lists = [(0, min(128, T))]
    while t_lists[-1][0] + t_lists[-1][1] < T:
        lo = t_lists[-1][0] + t_lists[-1][1]
        t_lists.append((lo, min(128, T - lo)))

    @functools.partial(
        pl.kernel,
        mesh=plsc.VectorSubcoreMesh(core_axis_name="c", subcore_axis_name="s"),
        compiler_params=pltpu.CompilerParams(use_tc_tiling_on_sc=False),
        out_type=jax.ShapeDtypeStruct((S, T, _EMBED), jnp.float32),
        scratch_types=[
            pltpu.VMEM((s_per_w, T), jnp.int32),
            pltpu.VMEM((_NBUF, T, _EMBED), jnp.float32),
            pltpu.SemaphoreType.DMA((_NBUF,)),
            pltpu.SemaphoreType.DMA((_NBUF,)),
        ],
    )
    def _emb(tok_hbm, table_hbm, out_hbm, idx_v, rows_v, gsem, osem):
        wid = lax.axis_index("s") * _NC + lax.axis_index("c")
        sbase = wid * s_per_w

        pltpu.sync_copy(tok_hbm.at[pl.ds(sbase, s_per_w)], idx_v)

        def start_gather(i, b):
            for lo, ln in t_lists:
                pltpu.async_copy(
                    table_hbm.at[idx_v.at[i, pl.ds(lo, ln)]],
                    rows_v.at[b, pl.ds(lo, ln)],
                    gsem.at[b],
                )

        def wait_gather(i, b):
            for lo, ln in t_lists:
                pltpu.make_async_copy(
                    table_hbm.at[idx_v.at[i, pl.ds(lo, ln)]],
                    rows_v.at[b, pl.ds(lo, ln)],
                    gsem.at[b],
                ).wait()

        def wait_scatter(b):
            pltpu.make_async_copy(
                rows_v.at[b], out_hbm.at[0], osem.at[b]
            ).wait()

        for i in range(_AHEAD):
            start_gather(i, i % _NBUF)

        def row_iter(i, carry):
            b = lax.rem(i, _NBUF)
            ia = i + _AHEAD
            ba = lax.rem(ia, _NBUF)

            @pl.when(i >= _NBUF - _AHEAD)
            def _():
                wait_scatter(ba)

            @pl.when(ia < s_per_w)
            def _():
                start_gather(ia, ba)

            wait_gather(i, b)

            def vec_body(v, carry2):
                for k in range(_ROW_UNROLL):
                    r = v * _ROW_UNROLL + k
                    rows_v[b, r // _VECS_PER_ROW,
                           pl.ds((r % _VECS_PER_ROW) * _L, _L)] = (
                        rows_v[b, r // _VECS_PER_ROW,
                               pl.ds((r % _VECS_PER_ROW) * _L, _L)] * _SCALE
                    )
                return carry2

            lax.fori_loop(0, T * _VECS_PER_ROW // _ROW_UNROLL, vec_body, 0)

            pltpu.async_copy(rows_v.at[b], out_hbm.at[sbase + i], osem.at[b])
            return carry

        lax.fori_loop(0, s_per_w, row_iter, 0)

        for i in range(s_per_w - (_NBUF - _AHEAD), s_per_w):
            wait_scatter(i % _NBUF)

    out = _emb(tok, table)
    return out


# speedup vs baseline: 1.2858x; 1.2858x over previous
import functools

import jax
import jax.numpy as jnp
from jax import lax
from jax.experimental import pallas as pl
from jax.experimental.pallas import tpu as pltpu
from jax.experimental.pallas import tpu_sc as plsc

_EMBED = 64
_PAD = 128
_SCALE = 8.0

_info = plsc.get_sparse_core_info()
_NC = _info.num_cores
_NS = _info.num_subcores
_L = _info.num_lanes
_NW = _NC * _NS

_CHUNK = 128
_VECS_PER_ROW = _EMBED // _L
_NBUF = 4
_AHEAD = 2
_ROW_UNROLL = 8


def kernel(tokens, table):
    B = tokens.shape[0] * tokens.shape[1]
    n_chunks_total = B // _CHUNK
    n_chunks = n_chunks_total // _NW
    tok2d = tokens.reshape((n_chunks_total, _CHUNK)).astype(jnp.int32)

    @functools.partial(
        pl.kernel,
        mesh=plsc.VectorSubcoreMesh(core_axis_name="c", subcore_axis_name="s"),
        compiler_params=pltpu.CompilerParams(use_tc_tiling_on_sc=False),
        out_type=jax.ShapeDtypeStruct((B, _PAD), jnp.float32),
        scratch_types=[
            pltpu.VMEM((n_chunks, _CHUNK), jnp.int32),
            pltpu.VMEM((_NBUF, _CHUNK, _EMBED), jnp.float32),
            pltpu.VMEM((_NBUF, _CHUNK, _PAD), jnp.float32),
            pltpu.SemaphoreType.DMA((_NBUF,)),
            pltpu.SemaphoreType.DMA((_NBUF,)),
        ],
    )
    def _emb(tok_hbm, table_hbm, out_hbm, idx_v, raw_v, pad_v, gsem, osem):
        wid = lax.axis_index("s") * _NC + lax.axis_index("c")
        cbase = wid * n_chunks

        pltpu.sync_copy(tok_hbm.at[pl.ds(cbase, n_chunks)], idx_v)

        def start_gather(c, b):
            pltpu.async_copy(
                table_hbm.at[idx_v.at[c]], raw_v.at[b], gsem.at[b]
            )

        for c in range(_AHEAD):
            start_gather(c, c % _NBUF)

        def chunk_body(c, carry):
            b = lax.rem(c, _NBUF)
            ca = c + _AHEAD
            ba = lax.rem(ca, _NBUF)

            @pl.when(c >= _NBUF - _AHEAD)
            def _():
                pltpu.make_async_copy(
                    pad_v.at[ba], out_hbm.at[pl.ds(0, _CHUNK)], osem.at[ba]
                ).wait()

            @pl.when(ca < n_chunks)
            def _():
                start_gather(ca, ba)

            pltpu.make_async_copy(
                table_hbm.at[idx_v.at[c]], raw_v.at[b], gsem.at[b]
            ).wait()

            def row_body(i, carry2):
                for k in range(_ROW_UNROLL):
                    r = i * _ROW_UNROLL + k
                    row = r // _VECS_PER_ROW
                    v = r % _VECS_PER_ROW
                    pad_v[b, row, pl.ds(v * _L, _L)] = (
                        raw_v[b, row, pl.ds(v * _L, _L)] * _SCALE
                    )
                return carry2

            lax.fori_loop(0, _CHUNK * _VECS_PER_ROW // _ROW_UNROLL,
                          row_body, 0)

            pltpu.async_copy(
                pad_v.at[b], out_hbm.at[pl.ds((cbase + c) * _CHUNK, _CHUNK)],
                osem.at[b],
            )
            return carry

        lax.fori_loop(0, n_chunks, chunk_body, 0)

        for c in range(n_chunks - (_NBUF - _AHEAD), n_chunks):
            b = c % _NBUF
            pltpu.make_async_copy(
                pad_v.at[b], out_hbm.at[pl.ds(0, _CHUNK)], osem.at[b]
            ).wait()

    out = _emb(tok2d, table)
    return out[:, :_EMBED].reshape(tokens.shape + (_EMBED,))


# speedup vs baseline: 1.6046x; 1.2480x over previous
import functools

import jax
import jax.numpy as jnp
from jax import lax
from jax.experimental import pallas as pl
from jax.experimental.pallas import tpu as pltpu
from jax.experimental.pallas import tpu_sc as plsc

_EMBED = 64
_PAD = 128
_SCALE = 8.0

_info = plsc.get_sparse_core_info()
_NC = _info.num_cores
_NS = _info.num_subcores
_L = _info.num_lanes
_NW = _NC * _NS

_CHUNK = 128
_VECS_PER_ROW = _EMBED // _L
_NBUF = 4
_AHEAD = 2
_ROW_UNROLL = 8


def kernel(tokens, table):
    B = tokens.shape[0] * tokens.shape[1]
    n_chunks_total = B // _CHUNK
    n_chunks = n_chunks_total // _NW
    tok2d = tokens.reshape((n_chunks_total, _CHUNK)).astype(jnp.int32)
    tpad = jnp.pad(table, ((0, 0), (0, _PAD - _EMBED)))

    @functools.partial(
        pl.kernel,
        mesh=plsc.VectorSubcoreMesh(core_axis_name="c", subcore_axis_name="s"),
        compiler_params=pltpu.CompilerParams(use_tc_tiling_on_sc=False),
        out_type=jax.ShapeDtypeStruct((B, _PAD), jnp.float32),
        scratch_types=[
            pltpu.VMEM((n_chunks, _CHUNK), jnp.int32),
            pltpu.VMEM((_NBUF, _CHUNK, _PAD), jnp.float32),
            pltpu.SemaphoreType.DMA((_NBUF,)),
            pltpu.SemaphoreType.DMA((_NBUF,)),
        ],
    )
    def _emb(tok_hbm, table_hbm, out_hbm, idx_v, rows_v, gsem, osem):
        wid = lax.axis_index("s") * _NC + lax.axis_index("c")
        cbase = wid * n_chunks

        pltpu.sync_copy(tok_hbm.at[pl.ds(cbase, n_chunks)], idx_v)

        def start_gather(c, b):
            pltpu.async_copy(
                table_hbm.at[idx_v.at[c]], rows_v.at[b], gsem.at[b]
            )

        for c in range(_AHEAD):
            start_gather(c, c % _NBUF)

        def chunk_body(c, carry):
            b = lax.rem(c, _NBUF)
            ca = c + _AHEAD
            ba = lax.rem(ca, _NBUF)

            @pl.when(c >= _NBUF - _AHEAD)
            def _():
                pltpu.make_async_copy(
                    rows_v.at[ba], out_hbm.at[pl.ds(0, _CHUNK)], osem.at[ba]
                ).wait()

            @pl.when(ca < n_chunks)
            def _():
                start_gather(ca, ba)

            pltpu.make_async_copy(
                table_hbm.at[idx_v.at[c]], rows_v.at[b], gsem.at[b]
            ).wait()

            def row_body(i, carry2):
                for k in range(_ROW_UNROLL):
                    r = i * _ROW_UNROLL + k
                    row = r // _VECS_PER_ROW
                    v = r % _VECS_PER_ROW
                    rows_v[b, row, pl.ds(v * _L, _L)] = (
                        rows_v[b, row, pl.ds(v * _L, _L)] * _SCALE
                    )
                return carry2

            lax.fori_loop(0, _CHUNK * _VECS_PER_ROW // _ROW_UNROLL,
                          row_body, 0)

            pltpu.async_copy(
                rows_v.at[b], out_hbm.at[pl.ds((cbase + c) * _CHUNK, _CHUNK)],
                osem.at[b],
            )
            return carry

        lax.fori_loop(0, n_chunks, chunk_body, 0)

        for c in range(n_chunks - (_NBUF - _AHEAD), n_chunks):
            b = c % _NBUF
            pltpu.make_async_copy(
                rows_v.at[b], out_hbm.at[pl.ds(0, _CHUNK)], osem.at[b]
            ).wait()

    out = _emb(tok2d, tpad)
    return out[:, :_EMBED].reshape(tokens.shape + (_EMBED,))
